# baseline (speedup 1.0000x reference)
import functools

import jax
import jax.numpy as jnp
import numpy as np
from jax import lax
from jax.experimental import pallas as pl
from jax.experimental.pallas import tpu as pltpu
from jax.experimental.pallas import tpu_sc as plsc

_NH = 8
_FC = 128


def _full_spec(shape):
    nd = len(shape)
    return pl.BlockSpec(shape, lambda *a, _nd=nd: (0,) * _nd)


def _ln(x, g, b):
    m = jnp.mean(x, axis=-1, keepdims=True)
    v = jnp.mean((x - m) ** 2, axis=-1, keepdims=True)
    return (x - m) * jax.lax.rsqrt(v + 1e-5) * g + b


def _dot(a, b):
    return jnp.dot(a, b, preferred_element_type=jnp.float32)


def _dotb(a, b):
    return jnp.dot(a.astype(jnp.bfloat16), b.astype(jnp.bfloat16),
                   preferred_element_type=jnp.float32)


def _encode_kernel(xs_ref, dp_ref, kn_ref, vn_ref,
                   wd1_ref, bd1_ref, wd2_ref, bd2_ref,
                   wp1_ref, bp1_ref, wp2_ref, bp2_ref,
                   we1_ref, be1_ref, we2_ref, be2_ref,
                   wq_ref, bq_ref, wk_ref, bk_ref, wv_ref, bv_ref,
                   wo_ref, bo_ref, ln1g_ref, ln1b_ref,
                   fw1_ref, fb1_ref, fw2_ref, fb2_ref,
                   ln2g_ref, ln2b_ref, o_ref):
    xs = xs_ref[...]
    B, D = xs.shape
    K = kn_ref.shape[1]
    dh = D // _NH

    deg = dp_ref[:, 0:1]
    prk = dp_ref[:, 1:2]
    de = _dotb(jax.nn.relu(deg * wd1_ref[...] + bd1_ref[...]), wd2_ref[...]) + bd2_ref[...]
    pe = _dotb(jax.nn.relu(prk * wp1_ref[...] + bp1_ref[...]), wp2_ref[...]) + bp2_ref[...]

    we1 = we1_ref[...]
    h = (_dotb(xs, we1[0:D]) + _dotb(de, we1[D:2 * D])
         + _dotb(pe, we1[2 * D:3 * D]) + be1_ref[...])
    xe = _dotb(jax.nn.relu(h), we2_ref[...]) + be2_ref[...]

    q0 = _dot(xe, wq_ref[...]) + bq_ref[...]
    k0 = _dot(xe, wk_ref[...]) + bk_ref[...]
    v0 = _dot(xe, wv_ref[...]) + bv_ref[...]
    kn = kn_ref[...].astype(jnp.float32)
    vn = vn_ref[...].astype(jnp.float32)

    hm = (jax.lax.broadcasted_iota(jnp.int32, (D, _NH), 0) // dh
          == jax.lax.broadcasted_iota(jnp.int32, (D, _NH), 1)).astype(jnp.float32)

    scale = 1.0 / np.sqrt(dh)
    prod = kn * q0[:, None, :]
    sn = _dot(prod.reshape(B * K, D), hm).reshape(B, K, _NH) * scale
    s0 = _dot(q0 * k0, hm) * scale
    m = jnp.maximum(jnp.max(sn, axis=1), s0)
    en = jnp.exp(sn - m[:, None, :])
    e0 = jnp.exp(s0 - m)
    den = e0 + jnp.sum(en, axis=1)
    en_x = _dot(en.reshape(B * K, _NH), hm.T).reshape(B, K, D)
    num = jnp.sum(en_x * vn, axis=1) + _dot(e0, hm.T) * v0
    ctx = num / _dot(den, hm.T)
    att = _dot(ctx, wo_ref[...]) + bo_ref[...]

    x1 = _ln(xe + att, ln1g_ref[...], ln1b_ref[...])
    ff = _dotb(jax.nn.relu(_dotb(x1, fw1_ref[...]) + fb1_ref[...]), fw2_ref[...]) + fb2_ref[...]
    o_ref[...] = _ln(x1 + ff, ln2g_ref[...], ln2b_ref[...])


def _kv_kernel(x_ref, wk_ref, bk_ref, wv_ref, bv_ref, ko_ref, vo_ref):
    x = x_ref[...]
    ko_ref[...] = _dot(x, wk_ref[...]) + bk_ref[...]
    vo_ref[...] = _dot(x, wv_ref[...]) + bv_ref[...]


def _z1_kernel(x_ref, w_ref, di_ref, o_ref):
    o_ref[0] = _dotb(x_ref[...], w_ref[...]) * di_ref[...]


def _mid_kernel(p_ref, z_ref, di_ref, w_ref, o_ref, *, nf):
    di = di_ref[...]
    cols = [p_ref[f] + z_ref[f] for f in range(nf)]
    h = jnp.concatenate(cols, axis=-1) * di
    h = jnp.where(h > 0, h, jnp.exp(jnp.minimum(h, 0.0)) - 1.0)
    o_ref[0] = _dotb(h, w_ref[...]) * di


def _towers_kernel(p_ref, z_ref, di_ref,
                   f1w_ref, f1b_ref, f2w_ref, f2b_ref,
                   g1w_ref, g1b_ref, g2w_ref, g2b_ref,
                   tf_ref, tg_ref, *, nf):
    di = di_ref[...]
    cols = [p_ref[f] + z_ref[f] for f in range(nf)]
    emb = jnp.concatenate(cols, axis=-1) * di

    def lrelu(x):
        return jnp.where(x > 0, x, 0.01 * x)

    a = lrelu(_dotb(emb, f1w_ref[...]) + f1b_ref[...])
    tf_ref[...] = lrelu(_dotb(a, f2w_ref[...]) + f2b_ref[...])
    b = lrelu(_dotb(emb, g1w_ref[...]) + g1b_ref[...])
    tg_ref[...] = lrelu(_dotb(b, g2w_ref[...]) + g2b_ref[...])


def _pred_kernel(a_ref, b_ref, o_ref):
    o_ref[...] = jnp.sum(a_ref[...] * b_ref[...], axis=1, keepdims=True)


def _sc_info():
    info = plsc.get_sparse_core_info()
    return info.num_cores, info.num_subcores


def _seg_plan(E, ns):
    epw = E // ns
    for g in range(128, 0, -8):
        for k in (5, 4, 3, 2, 1):
            if k * g <= 350 and epw % (g * k) == 0:
                return epw, g, k, epw // (g * k)
    return epw, 0, 0, 0


def _sc_segsum(zflat, er, zeros, *, nt, nf, fc, g, k, n_body):
    NC, NS = _sc_info()
    rps = nt // NS
    mesh = plsc.VectorSubcoreMesh(core_axis_name="c", subcore_axis_name="s")

    @functools.partial(
        pl.kernel, mesh=mesh,
        out_type=jax.ShapeDtypeStruct((nf, nt, fc), jnp.float32),
        scratch_types=[
            pltpu.VMEM((2 * k, g), jnp.int32),
            pltpu.VMEM((k, g, fc), jnp.float32),
            pltpu.VMEM_SHARED((nt, fc), jnp.float32),
            pltpu.SemaphoreType.DMA,
            pltpu.SemaphoreType.DMA,
        ],
    )
    def seg(z_hbm, er_hbm, zero_hbm, out_hbm, ev, rows, acc, sem_g, sem_s):
        cid = lax.axis_index("c")
        sid = lax.axis_index("s")
        for f in range(nf):
            @pl.when((f % NC) == cid)
            def _():
                pltpu.sync_copy(zero_hbm.at[pl.ds(sid * rps, rps)],
                                acc.at[pl.ds(sid * rps, rps)])
                plsc.subcore_barrier()

                def body(i, carry):
                    pltpu.sync_copy(er_hbm.at[f, sid, i], ev)
                    hg = []
                    for s in range(k):
                        hg.append(pltpu.async_copy(
                            z_hbm.at[ev.at[2 * s]], rows.at[s], sem_g))
                    hs = []
                    for s in range(k):
                        hg[s].wait()
                        hs.append(pltpu.async_copy(
                            rows.at[s], acc.at[ev.at[2 * s + 1]], sem_s,
                            add=True))
                    for h in hs:
                        h.wait()
                    return carry

                lax.fori_loop(0, n_body, body, 0)
                plsc.subcore_barrier()
                pltpu.sync_copy(acc.at[pl.ds(sid * rps, rps)],
                                out_hbm.at[f, pl.ds(sid * rps, rps)])

    return seg(zflat, er, zeros)


def _gather_plan(B, nw):
    bpw = B // nw
    for g in range(128, 0, -8):
        if bpw % g == 0:
            return bpw, g, bpw // g
    return bpw, 0, 0


def _sc_gather_pair(tab1, tab2, idx1, idx2):
    NC, NS = _sc_info()
    NW = NC * NS
    B = idx1.shape[0]
    D1, D2 = tab1.shape[1], tab2.shape[1]
    bpw, g, n_iter = _gather_plan(B, NW)
    dt1, dt2 = tab1.dtype, tab2.dtype
    mesh = plsc.VectorSubcoreMesh(core_axis_name="c", subcore_axis_name="s")

    @functools.partial(
        pl.kernel, mesh=mesh,
        out_type=[jax.ShapeDtypeStruct((B, D1), dt1),
                  jax.ShapeDtypeStruct((B, D2), dt2)],
        scratch_types=[
            pltpu.VMEM((g,), jnp.int32),
            pltpu.VMEM((g,), jnp.int32),
            pltpu.VMEM((g, D1), dt1),
            pltpu.VMEM((g, D2), dt2),
            pltpu.SemaphoreType.DMA,
            pltpu.SemaphoreType.DMA,
        ],
    )
    def gk(t1, t2, i1_hbm, i2_hbm, o1, o2, iv1, iv2, r1, r2, sem1, sem2):
        wid = lax.axis_index("s") * NC + lax.axis_index("c")

        def body(it, carry):
            base = wid * bpw + it * g
            pltpu.sync_copy(i1_hbm.at[pl.ds(base, g)], iv1)
            pltpu.sync_copy(i2_hbm.at[pl.ds(base, g)], iv2)
            h1 = pltpu.async_copy(t1.at[iv1], r1, sem1)
            h2 = pltpu.async_copy(t2.at[iv2], r2, sem2)
            h1.wait()
            pltpu.sync_copy(r1, o1.at[pl.ds(base, g)])
            h2.wait()
            pltpu.sync_copy(r2, o2.at[pl.ds(base, g)])
            return carry

        lax.fori_loop(0, n_iter, body, 0)

    return gk(tab1, tab2, idx1, idx2)


def _sc_gather_one(tab, idx):
    NC, NS = _sc_info()
    NW = NC * NS
    B = idx.shape[0]
    Dt = tab.shape[1]
    bpw, g, n_iter = _gather_plan(B, NW)
    mesh = plsc.VectorSubcoreMesh(core_axis_name="c", subcore_axis_name="s")

    @functools.partial(
        pl.kernel, mesh=mesh,
        out_type=jax.ShapeDtypeStruct((B, Dt), jnp.float32),
        scratch_types=[
            pltpu.VMEM((g,), jnp.int32),
            pltpu.VMEM((g, Dt), jnp.float32),
            pltpu.SemaphoreType.DMA,
        ],
    )
    def gk(t, i_hbm, o, iv, r, sem):
        wid = lax.axis_index("s") * NC + lax.axis_index("c")

        def body(it, carry):
            base = wid * bpw + it * g
            pltpu.sync_copy(i_hbm.at[pl.ds(base, g)], iv)
            pltpu.async_copy(t.at[iv], r, sem).wait()
            pltpu.sync_copy(r, o.at[pl.ds(base, g)])
            return carry

        lax.fori_loop(0, n_iter, body, 0)

    return gk(tab, idx)


def _row2(p, name):
    w = p[name]
    return w.reshape(1, -1) if w.ndim == 1 else w


def _pack_edges(src, dst, nt, nf, ns, g, k, n_body):
    s_r = src.reshape(ns, n_body, k, 1, g)
    d_r = dst.reshape(ns, n_body, k, 1, g)
    per_f = [jnp.concatenate([s_r + f * nt, d_r], axis=3) for f in range(nf)]
    return jnp.stack(per_f).reshape(nf, ns, n_body, 2 * k, g)


def kernel(X, edge_index, train_sample, top_k_indices, degrees, page_ranks, params):
    p = params
    N, D = X.shape
    K = top_k_indices.shape[1]
    nl = jnp.concatenate([train_sample[:, 0], train_sample[:, 1]])
    Nl = nl.shape[0]
    B = 128 if Nl % 128 == 0 else Nl
    BM = 1000 if N % 1000 == 0 else N
    NT = ((N + 127) // 128) * 128

    K_all, V_all = pl.pallas_call(
        _kv_kernel,
        grid=(N // BM,),
        in_specs=[pl.BlockSpec((BM, D), lambda i: (i, 0)),
                  _full_spec((D, D)), _full_spec((1, D)),
                  _full_spec((D, D)), _full_spec((1, D))],
        out_specs=[pl.BlockSpec((BM, D), lambda i: (i, 0)),
                   pl.BlockSpec((BM, D), lambda i: (i, 0))],
        out_shape=[jax.ShapeDtypeStruct((N, D), jnp.float32),
                   jax.ShapeDtypeStruct((N, D), jnp.float32)],
    )(X, p['wk'], _row2(p, 'bk'), p['wv'], _row2(p, 'bv'))

    dp = jnp.stack([degrees[nl], page_ranks[nl]], axis=1)
    fl = top_k_indices[nl].reshape(-1)
    kn_f, vn_f = _sc_gather_pair(K_all, V_all, fl, fl)
    kn = kn_f.reshape(Nl, K, D)
    vn = vn_f.reshape(Nl, K, D)
    Xs = _sc_gather_one(X, nl)

    w_names = ['deg_w1', 'deg_b1', 'deg_w2', 'deg_b2',
               'pr_w1', 'pr_b1', 'pr_w2', 'pr_b2',
               'pe_w1', 'pe_b1', 'pe_w2', 'pe_b2',
               'wq', 'bq', 'wk', 'bk', 'wv', 'bv', 'wo', 'bo',
               'ln1_g', 'ln1_b', 'ff_w1', 'ff_b1', 'ff_w2', 'ff_b2',
               'ln2_g', 'ln2_b']
    w_vals = [_row2(p, n) for n in w_names]
    cls = pl.pallas_call(
        _encode_kernel,
        grid=(Nl // B,),
        in_specs=[pl.BlockSpec((B, D), lambda i: (i, 0)),
                  pl.BlockSpec((B, 2), lambda i: (i, 0)),
                  pl.BlockSpec((B, K, D), lambda i: (i, 0, 0)),
                  pl.BlockSpec((B, K, D), lambda i: (i, 0, 0))]
                 + [_full_spec(w.shape) for w in w_vals],
        out_specs=pl.BlockSpec((B, D), lambda i: (i, 0)),
        out_shape=jax.ShapeDtypeStruct((Nl, D), jnp.float32),
    )(Xs, dp, kn, vn, *w_vals)

    Xn = X.at[nl].set(cls)

    src, dst = edge_index[0], edge_index[1]
    E = src.shape[0]
    deg = jax.ops.segment_sum(jnp.ones((E,), jnp.float32), dst, num_segments=N) + 1.0
    dinv = jax.lax.rsqrt(deg)
    di2 = dinv[:, None]

    H1 = p['gcn1_w'].shape[1]
    H2 = p['gcn2_w'].shape[1]
    NF1 = H1 // _FC
    NF2 = H2 // _FC
    zeros_acc = jnp.zeros((NT, _FC), jnp.float32)
    NC, NS = _sc_info()
    epw, g, kk, n_body = _seg_plan(E, NS)
    er1 = _pack_edges(src, dst, NT, NF1, NS, g, kk, n_body)

    z1 = pl.pallas_call(
        _z1_kernel,
        grid=(N // BM, NF1),
        in_specs=[pl.BlockSpec((BM, D), lambda i, f: (i, 0)),
                  pl.BlockSpec((D, _FC), lambda i, f: (0, f)),
                  pl.BlockSpec((BM, 1), lambda i, f: (i, 0))],
        out_specs=pl.BlockSpec((1, BM, _FC), lambda i, f: (f, i, 0)),
        out_shape=jax.ShapeDtypeStruct((NF1, NT, _FC), jnp.float32),
    )(Xn, p['gcn1_w'], di2)

    p1 = _sc_segsum(z1.reshape(NF1 * NT, _FC), er1, zeros_acc,
                    nt=NT, nf=NF1, fc=_FC, g=g, k=kk, n_body=n_body)

    z2 = pl.pallas_call(
        functools.partial(_mid_kernel, nf=NF1),
        grid=(N // BM, NF2),
        in_specs=[pl.BlockSpec((NF1, BM, _FC), lambda i, f: (0, i, 0)),
                  pl.BlockSpec((NF1, BM, _FC), lambda i, f: (0, i, 0)),
                  pl.BlockSpec((BM, 1), lambda i, f: (i, 0)),
                  pl.BlockSpec((H1, _FC), lambda i, f: (0, f))],
        out_specs=pl.BlockSpec((1, BM, _FC), lambda i, f: (f, i, 0)),
        out_shape=jax.ShapeDtypeStruct((NF2, NT, _FC), jnp.float32),
    )(p1, z1, di2, p['gcn2_w'])

    er2 = er1 if NF2 == NF1 else _pack_edges(src, dst, NT, NF2, NS, g, kk, n_body)
    p2 = _sc_segsum(z2.reshape(NF2 * NT, _FC), er2, zeros_acc,
                    nt=NT, nf=NF2, fc=_FC, g=g, k=kk, n_body=n_body)

    H3 = p['tf1_w'].shape[1]
    OUT = p['tf2_w'].shape[1]
    tf, tg = pl.pallas_call(
        functools.partial(_towers_kernel, nf=NF2),
        grid=(N // BM,),
        in_specs=[pl.BlockSpec((NF2, BM, _FC), lambda i: (0, i, 0)),
                  pl.BlockSpec((NF2, BM, _FC), lambda i: (0, i, 0)),
                  pl.BlockSpec((BM, 1), lambda i: (i, 0)),
                  _full_spec((H2, H3)), _full_spec((1, H3)),
                  _full_spec((H3, OUT)), _full_spec((1, OUT)),
                  _full_spec((H2, H3)), _full_spec((1, H3)),
                  _full_spec((H3, OUT)), _full_spec((1, OUT))],
        out_specs=[pl.BlockSpec((BM, OUT), lambda i: (i, 0)),
                   pl.BlockSpec((BM, OUT), lambda i: (i, 0))],
        out_shape=[jax.ShapeDtypeStruct((N, OUT), jnp.float32),
                   jax.ShapeDtypeStruct((N, OUT), jnp.float32)],
    )(p2, z2, di2,
      p['tf1_w'], _row2(p, 'tf1_b'), p['tf2_w'], _row2(p, 'tf2_b'),
      p['tg1_w'], _row2(p, 'tg1_b'), p['tg2_w'], _row2(p, 'tg2_b'))

    ia = jnp.asarray(train_sample[:, 0])
    ib = jnp.asarray(train_sample[:, 1])
    tfa, tgb = _sc_gather_pair(tf, tg, ia, ib)
    T = ia.shape[0]
    BT = 1024 if T % 1024 == 0 else T
    pred = pl.pallas_call(
        _pred_kernel,
        grid=(T // BT,),
        in_specs=[pl.BlockSpec((BT, OUT), lambda i: (i, 0)),
                  pl.BlockSpec((BT, OUT), lambda i: (i, 0))],
        out_specs=pl.BlockSpec((BT, 1), lambda i: (i, 0)),
        out_shape=jax.ShapeDtypeStruct((T, 1), jnp.float32),
    )(tfa, tgb)
    return pred


# speedup vs baseline: 1.1491x; 1.1491x over previous
import functools

import jax
import jax.numpy as jnp
import numpy as np
from jax import lax
from jax.experimental import pallas as pl
from jax.experimental.pallas import tpu as pltpu
from jax.experimental.pallas import tpu_sc as plsc

_NH = 8
_FC = 128


def _full_spec(shape):
    nd = len(shape)
    return pl.BlockSpec(shape, lambda *a, _nd=nd: (0,) * _nd)


def _ln(x, g, b):
    m = jnp.mean(x, axis=-1, keepdims=True)
    v = jnp.mean((x - m) ** 2, axis=-1, keepdims=True)
    return (x - m) * jax.lax.rsqrt(v + 1e-5) * g + b


def _dot(a, b):
    return jnp.dot(a, b, preferred_element_type=jnp.float32)


def _dotb(a, b):
    return jnp.dot(a.astype(jnp.bfloat16), b.astype(jnp.bfloat16),
                   preferred_element_type=jnp.float32)


def _encode_kernel(xs_ref, dp_ref, kn_ref, vn_ref,
                   wd1_ref, bd1_ref, wd2_ref, bd2_ref,
                   wp1_ref, bp1_ref, wp2_ref, bp2_ref,
                   we1_ref, be1_ref, we2_ref, be2_ref,
                   wq_ref, bq_ref, wk_ref, bk_ref, wv_ref, bv_ref,
                   wo_ref, bo_ref, ln1g_ref, ln1b_ref,
                   fw1_ref, fb1_ref, fw2_ref, fb2_ref,
                   ln2g_ref, ln2b_ref, o_ref):
    xs = xs_ref[...]
    B, D = xs.shape
    K = kn_ref.shape[0] // B
    dh = D // _NH

    deg = dp_ref[:, 0:1]
    prk = dp_ref[:, 1:2]
    de = _dotb(jax.nn.relu(deg * wd1_ref[...] + bd1_ref[...]), wd2_ref[...]) + bd2_ref[...]
    pe = _dotb(jax.nn.relu(prk * wp1_ref[...] + bp1_ref[...]), wp2_ref[...]) + bp2_ref[...]

    we1 = we1_ref[...]
    h = (_dotb(xs, we1[0:D]) + _dotb(de, we1[D:2 * D])
         + _dotb(pe, we1[2 * D:3 * D]) + be1_ref[...])
    xe = _dotb(jax.nn.relu(h), we2_ref[...]) + be2_ref[...]

    q0 = _dot(xe, wq_ref[...]) + bq_ref[...]
    k0 = _dot(xe, wk_ref[...]) + bk_ref[...]
    v0 = _dot(xe, wv_ref[...]) + bv_ref[...]
    kn = kn_ref[...].astype(jnp.float32).reshape(B, K, D)
    vn = vn_ref[...].astype(jnp.float32).reshape(B, K, D)

    hm = (jax.lax.broadcasted_iota(jnp.int32, (D, _NH), 0) // dh
          == jax.lax.broadcasted_iota(jnp.int32, (D, _NH), 1)).astype(jnp.float32)

    scale = 1.0 / np.sqrt(dh)
    prod = kn * q0[:, None, :]
    sn = _dot(prod.reshape(B * K, D), hm).reshape(B, K, _NH) * scale
    s0 = _dot(q0 * k0, hm) * scale
    m = jnp.maximum(jnp.max(sn, axis=1), s0)
    en = jnp.exp(sn - m[:, None, :])
    e0 = jnp.exp(s0 - m)
    den = e0 + jnp.sum(en, axis=1)
    en_x = _dot(en.reshape(B * K, _NH), hm.T).reshape(B, K, D)
    num = jnp.sum(en_x * vn, axis=1) + _dot(e0, hm.T) * v0
    ctx = num / _dot(den, hm.T)
    att = _dot(ctx, wo_ref[...]) + bo_ref[...]

    x1 = _ln(xe + att, ln1g_ref[...], ln1b_ref[...])
    ff = _dotb(jax.nn.relu(_dotb(x1, fw1_ref[...]) + fb1_ref[...]), fw2_ref[...]) + fb2_ref[...]
    o_ref[...] = _ln(x1 + ff, ln2g_ref[...], ln2b_ref[...])


def _kv_kernel(x_ref, wk_ref, bk_ref, wv_ref, bv_ref, ko_ref, vo_ref):
    x = x_ref[...]
    ko_ref[...] = _dot(x, wk_ref[...]) + bk_ref[...]
    vo_ref[...] = _dot(x, wv_ref[...]) + bv_ref[...]


def _z1_kernel(x_ref, w_ref, di_ref, o_ref):
    o_ref[0] = _dotb(x_ref[...], w_ref[...]) * di_ref[...]


def _mid_kernel(p_ref, z_ref, di_ref, w_ref, o_ref, *, nf):
    di = di_ref[...]
    cols = [p_ref[f] + z_ref[f] for f in range(nf)]
    h = jnp.concatenate(cols, axis=-1) * di
    h = jnp.where(h > 0, h, jnp.exp(jnp.minimum(h, 0.0)) - 1.0)
    o_ref[0] = _dotb(h, w_ref[...]) * di


def _towers_kernel(p_ref, z_ref, di_ref,
                   f1w_ref, f1b_ref, f2w_ref, f2b_ref,
                   g1w_ref, g1b_ref, g2w_ref, g2b_ref,
                   tf_ref, tg_ref, *, nf):
    di = di_ref[...]
    cols = [p_ref[f] + z_ref[f] for f in range(nf)]
    emb = jnp.concatenate(cols, axis=-1) * di

    def lrelu(x):
        return jnp.where(x > 0, x, 0.01 * x)

    a = lrelu(_dotb(emb, f1w_ref[...]) + f1b_ref[...])
    tf_ref[...] = lrelu(_dotb(a, f2w_ref[...]) + f2b_ref[...])
    b = lrelu(_dotb(emb, g1w_ref[...]) + g1b_ref[...])
    tg_ref[...] = lrelu(_dotb(b, g2w_ref[...]) + g2b_ref[...])


def _pred_kernel(a_ref, b_ref, o_ref):
    o_ref[...] = jnp.sum(a_ref[...] * b_ref[...], axis=1, keepdims=True)


def _sc_info():
    info = plsc.get_sparse_core_info()
    return info.num_cores, info.num_subcores


def _seg_plan(E, ns):
    epw = E // ns
    for g in range(128, 0, -8):
        for k in (5, 4, 3, 2, 1):
            if k * g <= 350 and epw % (g * k) == 0:
                return epw, g, k, epw // (g * k)
    return epw, 0, 0, 0


def _sc_segsum(zflat, er, zeros, *, nt, nf, fc, g, k, n_body):
    NC, NS = _sc_info()
    rps = nt // NS
    mesh = plsc.VectorSubcoreMesh(core_axis_name="c", subcore_axis_name="s")

    @functools.partial(
        pl.kernel, mesh=mesh,
        out_type=jax.ShapeDtypeStruct((nf, nt, fc), jnp.float32),
        scratch_types=[
            pltpu.VMEM((2 * k, g), jnp.int32),
            pltpu.VMEM((k, g, fc), jnp.float32),
            pltpu.VMEM_SHARED((nt, fc), jnp.float32),
            pltpu.SemaphoreType.DMA,
            pltpu.SemaphoreType.DMA,
        ],
    )
    def seg(z_hbm, er_hbm, zero_hbm, out_hbm, ev, rows, acc, sem_g, sem_s):
        cid = lax.axis_index("c")
        sid = lax.axis_index("s")
        for f in range(nf):
            @pl.when((f % NC) == cid)
            def _():
                pltpu.sync_copy(zero_hbm.at[pl.ds(sid * rps, rps)],
                                acc.at[pl.ds(sid * rps, rps)])
                plsc.subcore_barrier()

                def body(i, carry):
                    pltpu.sync_copy(er_hbm.at[f, sid, i], ev)
                    hg = []
                    for s in range(k):
                        hg.append(pltpu.async_copy(
                            z_hbm.at[ev.at[2 * s]], rows.at[s], sem_g))
                    hs = []
                    for s in range(k):
                        hg[s].wait()
                        hs.append(pltpu.async_copy(
                            rows.at[s], acc.at[ev.at[2 * s + 1]], sem_s,
                            add=True))
                    for h in hs:
                        h.wait()
                    return carry

                lax.fori_loop(0, n_body, body, 0)
                plsc.subcore_barrier()
                pltpu.sync_copy(acc.at[pl.ds(sid * rps, rps)],
                                out_hbm.at[f, pl.ds(sid * rps, rps)])

    return seg(zflat, er, zeros)


def _gather_plan(B, nw):
    bpw = B // nw
    for g in range(128, 0, -8):
        if bpw % g == 0:
            return bpw, g, bpw // g
    return bpw, 0, 0


def _sc_gather_pair(tab1, tab2, idx1, idx2):
    NC, NS = _sc_info()
    NW = NC * NS
    B = idx1.shape[0]
    D1, D2 = tab1.shape[1], tab2.shape[1]
    bpw, g, n_iter = _gather_plan(B, NW)
    dt1, dt2 = tab1.dtype, tab2.dtype
    mesh = plsc.VectorSubcoreMesh(core_axis_name="c", subcore_axis_name="s")

    @functools.partial(
        pl.kernel, mesh=mesh,
        out_type=[jax.ShapeDtypeStruct((B, D1), dt1),
                  jax.ShapeDtypeStruct((B, D2), dt2)],
        scratch_types=[
            pltpu.VMEM((g,), jnp.int32),
            pltpu.VMEM((g,), jnp.int32),
            pltpu.VMEM((g, D1), dt1),
            pltpu.VMEM((g, D2), dt2),
            pltpu.SemaphoreType.DMA,
            pltpu.SemaphoreType.DMA,
        ],
    )
    def gk(t1, t2, i1_hbm, i2_hbm, o1, o2, iv1, iv2, r1, r2, sem1, sem2):
        wid = lax.axis_index("s") * NC + lax.axis_index("c")

        def body(it, carry):
            base = wid * bpw + it * g
            pltpu.sync_copy(i1_hbm.at[pl.ds(base, g)], iv1)
            pltpu.sync_copy(i2_hbm.at[pl.ds(base, g)], iv2)
            h1 = pltpu.async_copy(t1.at[iv1], r1, sem1)
            h2 = pltpu.async_copy(t2.at[iv2], r2, sem2)
            h1.wait()
            pltpu.sync_copy(r1, o1.at[pl.ds(base, g)])
            h2.wait()
            pltpu.sync_copy(r2, o2.at[pl.ds(base, g)])
            return carry

        lax.fori_loop(0, n_iter, body, 0)

    return gk(tab1, tab2, idx1, idx2)


def _sc_gather_one(tab, idx):
    NC, NS = _sc_info()
    NW = NC * NS
    B = idx.shape[0]
    Dt = tab.shape[1]
    bpw, g, n_iter = _gather_plan(B, NW)
    mesh = plsc.VectorSubcoreMesh(core_axis_name="c", subcore_axis_name="s")

    @functools.partial(
        pl.kernel, mesh=mesh,
        out_type=jax.ShapeDtypeStruct((B, Dt), jnp.float32),
        scratch_types=[
            pltpu.VMEM((g,), jnp.int32),
            pltpu.VMEM((g, Dt), jnp.float32),
            pltpu.SemaphoreType.DMA,
        ],
    )
    def gk(t, i_hbm, o, iv, r, sem):
        wid = lax.axis_index("s") * NC + lax.axis_index("c")

        def body(it, carry):
            base = wid * bpw + it * g
            pltpu.sync_copy(i_hbm.at[pl.ds(base, g)], iv)
            pltpu.async_copy(t.at[iv], r, sem).wait()
            pltpu.sync_copy(r, o.at[pl.ds(base, g)])
            return carry

        lax.fori_loop(0, n_iter, body, 0)

    return gk(tab, idx)


def _row2(p, name):
    w = p[name]
    return w.reshape(1, -1) if w.ndim == 1 else w


def _pack_edges(src, dst, nt, nf, ns, g, k, n_body):
    s_r = src.reshape(ns, n_body, k, 1, g)
    d_r = dst.reshape(ns, n_body, k, 1, g)
    per_f = [jnp.concatenate([s_r + f * nt, d_r], axis=3) for f in range(nf)]
    return jnp.stack(per_f).reshape(nf, ns, n_body, 2 * k, g)


def kernel(X, edge_index, train_sample, top_k_indices, degrees, page_ranks, params):
    p = params
    N, D = X.shape
    K = top_k_indices.shape[1]
    nl = jnp.concatenate([train_sample[:, 0], train_sample[:, 1]])
    Nl = nl.shape[0]
    B = 128 if Nl % 128 == 0 else Nl
    BM = 1000 if N % 1000 == 0 else N
    NT = ((N + 127) // 128) * 128

    K_all, V_all = pl.pallas_call(
        _kv_kernel,
        grid=(N // BM,),
        in_specs=[pl.BlockSpec((BM, D), lambda i: (i, 0)),
                  _full_spec((D, D)), _full_spec((1, D)),
                  _full_spec((D, D)), _full_spec((1, D))],
        out_specs=[pl.BlockSpec((BM, D), lambda i: (i, 0)),
                   pl.BlockSpec((BM, D), lambda i: (i, 0))],
        out_shape=[jax.ShapeDtypeStruct((N, D), jnp.float32),
                   jax.ShapeDtypeStruct((N, D), jnp.float32)],
    )(X, p['wk'], _row2(p, 'bk'), p['wv'], _row2(p, 'bv'))

    dp = jnp.stack([degrees[nl], page_ranks[nl]], axis=1)
    fl = top_k_indices[nl].reshape(-1)
    kn_f, vn_f = _sc_gather_pair(K_all, V_all, fl, fl)
    Xs = _sc_gather_one(X, nl)

    w_names = ['deg_w1', 'deg_b1', 'deg_w2', 'deg_b2',
               'pr_w1', 'pr_b1', 'pr_w2', 'pr_b2',
               'pe_w1', 'pe_b1', 'pe_w2', 'pe_b2',
               'wq', 'bq', 'wk', 'bk', 'wv', 'bv', 'wo', 'bo',
               'ln1_g', 'ln1_b', 'ff_w1', 'ff_b1', 'ff_w2', 'ff_b2',
               'ln2_g', 'ln2_b']
    w_vals = [_row2(p, n) for n in w_names]
    cls = pl.pallas_call(
        _encode_kernel,
        grid=(Nl // B,),
        in_specs=[pl.BlockSpec((B, D), lambda i: (i, 0)),
                  pl.BlockSpec((B, 2), lambda i: (i, 0)),
                  pl.BlockSpec((B * K, D), lambda i: (i, 0)),
                  pl.BlockSpec((B * K, D), lambda i: (i, 0))]
                 + [_full_spec(w.shape) for w in w_vals],
        out_specs=pl.BlockSpec((B, D), lambda i: (i, 0)),
        out_shape=jax.ShapeDtypeStruct((Nl, D), jnp.float32),
    )(Xs, dp, kn_f, vn_f, *w_vals)

    Xn = X.at[nl].set(cls)

    src, dst = edge_index[0], edge_index[1]
    E = src.shape[0]
    H1 = p['gcn1_w'].shape[1]
    H2 = p['gcn2_w'].shape[1]
    NF1 = H1 // _FC
    NF2 = H2 // _FC
    zeros_acc = jnp.zeros((NT, _FC), jnp.float32)
    NC, NS = _sc_info()
    epw, g, kk, n_body = _seg_plan(E, NS)
    er1 = _pack_edges(src, dst, NT, NF1, NS, g, kk, n_body)
    cnt = _sc_segsum(jnp.ones((NT, _FC), jnp.float32), er1[0:1], zeros_acc,
                     nt=NT, nf=1, fc=_FC, g=g, k=kk, n_body=n_body)
    deg = cnt[0, :N, 0] + 1.0
    dinv = jax.lax.rsqrt(deg)
    di2 = dinv[:, None]

    z1 = pl.pallas_call(
        _z1_kernel,
        grid=(N // BM, NF1),
        in_specs=[pl.BlockSpec((BM, D), lambda i, f: (i, 0)),
                  pl.BlockSpec((D, _FC), lambda i, f: (0, f)),
                  pl.BlockSpec((BM, 1), lambda i, f: (i, 0))],
        out_specs=pl.BlockSpec((1, BM, _FC), lambda i, f: (f, i, 0)),
        out_shape=jax.ShapeDtypeStruct((NF1, NT, _FC), jnp.float32),
    )(Xn, p['gcn1_w'], di2)

    p1 = _sc_segsum(z1.reshape(NF1 * NT, _FC), er1, zeros_acc,
                    nt=NT, nf=NF1, fc=_FC, g=g, k=kk, n_body=n_body)

    z2 = pl.pallas_call(
        functools.partial(_mid_kernel, nf=NF1),
        grid=(N // BM, NF2),
        in_specs=[pl.BlockSpec((NF1, BM, _FC), lambda i, f: (0, i, 0)),
                  pl.BlockSpec((NF1, BM, _FC), lambda i, f: (0, i, 0)),
                  pl.BlockSpec((BM, 1), lambda i, f: (i, 0)),
                  pl.BlockSpec((H1, _FC), lambda i, f: (0, f))],
        out_specs=pl.BlockSpec((1, BM, _FC), lambda i, f: (f, i, 0)),
        out_shape=jax.ShapeDtypeStruct((NF2, NT, _FC), jnp.float32),
    )(p1, z1, di2, p['gcn2_w'])

    er2 = er1 if NF2 == NF1 else _pack_edges(src, dst, NT, NF2, NS, g, kk, n_body)
    p2 = _sc_segsum(z2.reshape(NF2 * NT, _FC), er2, zeros_acc,
                    nt=NT, nf=NF2, fc=_FC, g=g, k=kk, n_body=n_body)

    H3 = p['tf1_w'].shape[1]
    OUT = p['tf2_w'].shape[1]
    tf, tg = pl.pallas_call(
        functools.partial(_towers_kernel, nf=NF2),
        grid=(N // BM,),
        in_specs=[pl.BlockSpec((NF2, BM, _FC), lambda i: (0, i, 0)),
                  pl.BlockSpec((NF2, BM, _FC), lambda i: (0, i, 0)),
                  pl.BlockSpec((BM, 1), lambda i: (i, 0)),
                  _full_spec((H2, H3)), _full_spec((1, H3)),
                  _full_spec((H3, OUT)), _full_spec((1, OUT)),
                  _full_spec((H2, H3)), _full_spec((1, H3)),
                  _full_spec((H3, OUT)), _full_spec((1, OUT))],
        out_specs=[pl.BlockSpec((BM, OUT), lambda i: (i, 0)),
                   pl.BlockSpec((BM, OUT), lambda i: (i, 0))],
        out_shape=[jax.ShapeDtypeStruct((N, OUT), jnp.float32),
                   jax.ShapeDtypeStruct((N, OUT), jnp.float32)],
    )(p2, z2, di2,
      p['tf1_w'], _row2(p, 'tf1_b'), p['tf2_w'], _row2(p, 'tf2_b'),
      p['tg1_w'], _row2(p, 'tg1_b'), p['tg2_w'], _row2(p, 'tg2_b'))

    ia = jnp.asarray(train_sample[:, 0])
    ib = jnp.asarray(train_sample[:, 1])
    tfa, tgb = _sc_gather_pair(tf, tg, ia, ib)
    T = ia.shape[0]
    BT = 1024 if T % 1024 == 0 else T
    pred = pl.pallas_call(
        _pred_kernel,
        grid=(T // BT,),
        in_specs=[pl.BlockSpec((BT, OUT), lambda i: (i, 0)),
                  pl.BlockSpec((BT, OUT), lambda i: (i, 0))],
        out_specs=pl.BlockSpec((BT, 1), lambda i: (i, 0)),
        out_shape=jax.ShapeDtypeStruct((T, 1), jnp.float32),
    )(tfa, tgb)
    return pred


# speedup vs baseline: 1.3374x; 1.1639x over previous
import functools

import jax
import jax.numpy as jnp
import numpy as np
from jax import lax
from jax.experimental import pallas as pl
from jax.experimental.pallas import tpu as pltpu
from jax.experimental.pallas import tpu_sc as plsc

_NH = 8
_FC = 128


def _full_spec(shape):
    nd = len(shape)
    return pl.BlockSpec(shape, lambda *a, _nd=nd: (0,) * _nd)


def _ln(x, g, b):
    m = jnp.mean(x, axis=-1, keepdims=True)
    v = jnp.mean((x - m) ** 2, axis=-1, keepdims=True)
    return (x - m) * jax.lax.rsqrt(v + 1e-5) * g + b


def _dot(a, b):
    return jnp.dot(a, b, preferred_element_type=jnp.float32)


def _dotb(a, b):
    return jnp.dot(a.astype(jnp.bfloat16), b.astype(jnp.bfloat16),
                   preferred_element_type=jnp.float32)


def _encode_kernel(xs_ref, dp_ref, kn_ref, vn_ref,
                   wd1_ref, bd1_ref, wd2_ref, bd2_ref,
                   wp1_ref, bp1_ref, wp2_ref, bp2_ref,
                   we1_ref, be1_ref, we2_ref, be2_ref,
                   wq_ref, bq_ref, wk_ref, bk_ref, wv_ref, bv_ref,
                   wo_ref, bo_ref, ln1g_ref, ln1b_ref,
                   fw1_ref, fb1_ref, fw2_ref, fb2_ref,
                   ln2g_ref, ln2b_ref, o_ref):
    xs = xs_ref[...]
    B, D = xs.shape
    K = kn_ref.shape[0] // B
    dh = D // _NH

    deg = dp_ref[:, 0:1]
    prk = dp_ref[:, 1:2]
    de = _dotb(jax.nn.relu(deg * wd1_ref[...] + bd1_ref[...]), wd2_ref[...]) + bd2_ref[...]
    pe = _dotb(jax.nn.relu(prk * wp1_ref[...] + bp1_ref[...]), wp2_ref[...]) + bp2_ref[...]

    we1 = we1_ref[...]
    h = (_dotb(xs, we1[0:D]) + _dotb(de, we1[D:2 * D])
         + _dotb(pe, we1[2 * D:3 * D]) + be1_ref[...])
    xe = _dotb(jax.nn.relu(h), we2_ref[...]) + be2_ref[...]

    q0 = _dot(xe, wq_ref[...]) + bq_ref[...]
    k0 = _dot(xe, wk_ref[...]) + bk_ref[...]
    v0 = _dot(xe, wv_ref[...]) + bv_ref[...]
    kn = kn_ref[...].astype(jnp.float32).reshape(B, K, D)
    vn = vn_ref[...].astype(jnp.float32).reshape(B, K, D)

    hm = (jax.lax.broadcasted_iota(jnp.int32, (D, _NH), 0) // dh
          == jax.lax.broadcasted_iota(jnp.int32, (D, _NH), 1)).astype(jnp.float32)

    scale = 1.0 / np.sqrt(dh)
    prod = kn * q0[:, None, :]
    sn = _dot(prod.reshape(B * K, D), hm).reshape(B, K, _NH) * scale
    s0 = _dot(q0 * k0, hm) * scale
    m = jnp.maximum(jnp.max(sn, axis=1), s0)
    en = jnp.exp(sn - m[:, None, :])
    e0 = jnp.exp(s0 - m)
    den = e0 + jnp.sum(en, axis=1)
    en_x = _dot(en.reshape(B * K, _NH), hm.T).reshape(B, K, D)
    num = jnp.sum(en_x * vn, axis=1) + _dot(e0, hm.T) * v0
    ctx = num / _dot(den, hm.T)
    att = _dot(ctx, wo_ref[...]) + bo_ref[...]

    x1 = _ln(xe + att, ln1g_ref[...], ln1b_ref[...])
    ff = _dotb(jax.nn.relu(_dotb(x1, fw1_ref[...]) + fb1_ref[...]), fw2_ref[...]) + fb2_ref[...]
    o_ref[...] = _ln(x1 + ff, ln2g_ref[...], ln2b_ref[...])


def _kv_kernel(x_ref, wk_ref, bk_ref, wv_ref, bv_ref, ko_ref, vo_ref):
    x = x_ref[...]
    ko_ref[...] = _dot(x, wk_ref[...]) + bk_ref[...]
    vo_ref[...] = _dot(x, wv_ref[...]) + bv_ref[...]


def _z1_kernel(x_ref, w_ref, di_ref, o_ref):
    o_ref[0] = _dotb(x_ref[...], w_ref[...]) * di_ref[...]


def _mid_kernel(p_ref, z_ref, di_ref, w_ref, o_ref, *, nf):
    di = di_ref[...]
    cols = [p_ref[f] + z_ref[f] for f in range(nf)]
    h = jnp.concatenate(cols, axis=-1) * di
    h = jnp.where(h > 0, h, jnp.exp(jnp.minimum(h, 0.0)) - 1.0)
    o_ref[0] = _dotb(h, w_ref[...]) * di


def _towers_kernel(p_ref, z_ref, di_ref,
                   f1w_ref, f1b_ref, f2w_ref, f2b_ref,
                   g1w_ref, g1b_ref, g2w_ref, g2b_ref,
                   tf_ref, tg_ref, *, nf):
    di = di_ref[...]
    cols = [p_ref[f] + z_ref[f] for f in range(nf)]
    emb = jnp.concatenate(cols, axis=-1) * di

    def lrelu(x):
        return jnp.where(x > 0, x, 0.01 * x)

    a = lrelu(_dotb(emb, f1w_ref[...]) + f1b_ref[...])
    tf_ref[...] = lrelu(_dotb(a, f2w_ref[...]) + f2b_ref[...])
    b = lrelu(_dotb(emb, g1w_ref[...]) + g1b_ref[...])
    tg_ref[...] = lrelu(_dotb(b, g2w_ref[...]) + g2b_ref[...])


def _pred_kernel(a_ref, b_ref, o_ref):
    o_ref[...] = jnp.sum(a_ref[...] * b_ref[...], axis=1, keepdims=True)


def _sc_info():
    info = plsc.get_sparse_core_info()
    return info.num_cores, info.num_subcores


def _seg_plan(E, ns):
    epw = E // ns
    for g in range(128, 0, -8):
        for k in (5, 4, 3, 2, 1):
            if k * g <= 350 and epw % (g * k) == 0:
                return epw, g, k, epw // (g * k)
    return epw, 0, 0, 0


def _sc_segsum(zflat, er, zeros, *, nt, nf, fc, g, k, n_body):
    NC, NS = _sc_info()
    rps = nt // NS
    mesh = plsc.VectorSubcoreMesh(core_axis_name="c", subcore_axis_name="s")

    @functools.partial(
        pl.kernel, mesh=mesh,
        out_type=jax.ShapeDtypeStruct((nf, nt, fc), jnp.float32),
        scratch_types=[
            pltpu.VMEM((2 * k, g), jnp.int32),
            pltpu.VMEM((k, g, fc), jnp.float32),
            pltpu.VMEM_SHARED((nt, fc), jnp.float32),
            pltpu.SemaphoreType.DMA,
            pltpu.SemaphoreType.DMA,
        ],
    )
    def seg(z_hbm, er_hbm, zero_hbm, out_hbm, ev, rows, acc, sem_g, sem_s):
        cid = lax.axis_index("c")
        sid = lax.axis_index("s")
        for f in range(nf):
            @pl.when((f % NC) == cid)
            def _():
                pltpu.sync_copy(zero_hbm.at[pl.ds(sid * rps, rps)],
                                acc.at[pl.ds(sid * rps, rps)])
                plsc.subcore_barrier()

                def body(i, carry):
                    pltpu.sync_copy(er_hbm.at[f, sid, i], ev)
                    hg = []
                    for s in range(k):
                        hg.append(pltpu.async_copy(
                            z_hbm.at[ev.at[2 * s]], rows.at[s], sem_g))
                    hs = []
                    for s in range(k):
                        hg[s].wait()
                        hs.append(pltpu.async_copy(
                            rows.at[s], acc.at[ev.at[2 * s + 1]], sem_s,
                            add=True))
                    for h in hs:
                        h.wait()
                    return carry

                lax.fori_loop(0, n_body, body, 0)
                plsc.subcore_barrier()
                pltpu.sync_copy(acc.at[pl.ds(sid * rps, rps)],
                                out_hbm.at[f, pl.ds(sid * rps, rps)])

    return seg(zflat, er, zeros)


def _sc_count(er, ones_gfc, zeros, *, nt, fc, g, k, n_body):
    NC, NS = _sc_info()
    rps = nt // NS
    mesh = plsc.VectorSubcoreMesh(core_axis_name="c", subcore_axis_name="s")

    @functools.partial(
        pl.kernel, mesh=mesh,
        out_type=jax.ShapeDtypeStruct((nt, fc), jnp.float32),
        scratch_types=[
            pltpu.VMEM((2 * k, g), jnp.int32),
            pltpu.VMEM((g, fc), jnp.float32),
            pltpu.VMEM_SHARED((nt, fc), jnp.float32),
            pltpu.SemaphoreType.DMA,
        ],
    )
    def ck(er_hbm, ones_hbm, zero_hbm, out_hbm, ev, onev, acc, sem_s):
        cid = lax.axis_index("c")
        sid = lax.axis_index("s")

        @pl.when(cid == 0)
        def _():
            pltpu.sync_copy(ones_hbm, onev)
            pltpu.sync_copy(zero_hbm.at[pl.ds(sid * rps, rps)],
                            acc.at[pl.ds(sid * rps, rps)])
            plsc.subcore_barrier()

            def body(i, carry):
                pltpu.sync_copy(er_hbm.at[0, sid, i], ev)
                hs = []
                for s in range(k):
                    hs.append(pltpu.async_copy(
                        onev, acc.at[ev.at[2 * s + 1]], sem_s, add=True))
                for h in hs:
                    h.wait()
                return carry

            lax.fori_loop(0, n_body, body, 0)
            plsc.subcore_barrier()
            pltpu.sync_copy(acc.at[pl.ds(sid * rps, rps)],
                            out_hbm.at[pl.ds(sid * rps, rps)])

    return ck(er, ones_gfc, zeros)


def _gather_plan(B, nw):
    bpw = B // nw
    for g in range(128, 0, -8):
        if bpw % g == 0:
            return bpw, g, bpw // g
    return bpw, 0, 0


def _sc_gather_pair(tab1, tab2, idx1, idx2):
    NC, NS = _sc_info()
    NW = NC * NS
    B = idx1.shape[0]
    D1, D2 = tab1.shape[1], tab2.shape[1]
    bpw, g, n_iter = _gather_plan(B, NW)
    dt1, dt2 = tab1.dtype, tab2.dtype
    mesh = plsc.VectorSubcoreMesh(core_axis_name="c", subcore_axis_name="s")

    @functools.partial(
        pl.kernel, mesh=mesh,
        out_type=[jax.ShapeDtypeStruct((B, D1), dt1),
                  jax.ShapeDtypeStruct((B, D2), dt2)],
        scratch_types=[
            pltpu.VMEM((g,), jnp.int32),
            pltpu.VMEM((g,), jnp.int32),
            pltpu.VMEM((g, D1), dt1),
            pltpu.VMEM((g, D2), dt2),
            pltpu.SemaphoreType.DMA,
            pltpu.SemaphoreType.DMA,
        ],
    )
    def gk(t1, t2, i1_hbm, i2_hbm, o1, o2, iv1, iv2, r1, r2, sem1, sem2):
        wid = lax.axis_index("s") * NC + lax.axis_index("c")

        def body(it, carry):
            base = wid * bpw + it * g
            pltpu.sync_copy(i1_hbm.at[pl.ds(base, g)], iv1)
            pltpu.sync_copy(i2_hbm.at[pl.ds(base, g)], iv2)
            h1 = pltpu.async_copy(t1.at[iv1], r1, sem1)
            h2 = pltpu.async_copy(t2.at[iv2], r2, sem2)
            h1.wait()
            pltpu.sync_copy(r1, o1.at[pl.ds(base, g)])
            h2.wait()
            pltpu.sync_copy(r2, o2.at[pl.ds(base, g)])
            return carry

        lax.fori_loop(0, n_iter, body, 0)

    return gk(tab1, tab2, idx1, idx2)


def _sc_gather_one(tab, idx):
    NC, NS = _sc_info()
    NW = NC * NS
    B = idx.shape[0]
    Dt = tab.shape[1]
    bpw, g, n_iter = _gather_plan(B, NW)
    mesh = plsc.VectorSubcoreMesh(core_axis_name="c", subcore_axis_name="s")

    @functools.partial(
        pl.kernel, mesh=mesh,
        out_type=jax.ShapeDtypeStruct((B, Dt), jnp.float32),
        scratch_types=[
            pltpu.VMEM((g,), jnp.int32),
            pltpu.VMEM((g, Dt), jnp.float32),
            pltpu.SemaphoreType.DMA,
        ],
    )
    def gk(t, i_hbm, o, iv, r, sem):
        wid = lax.axis_index("s") * NC + lax.axis_index("c")

        def body(it, carry):
            base = wid * bpw + it * g
            pltpu.sync_copy(i_hbm.at[pl.ds(base, g)], iv)
            pltpu.async_copy(t.at[iv], r, sem).wait()
            pltpu.sync_copy(r, o.at[pl.ds(base, g)])
            return carry

        lax.fori_loop(0, n_iter, body, 0)

    return gk(tab, idx)


def _row2(p, name):
    w = p[name]
    return w.reshape(1, -1) if w.ndim == 1 else w


def _pack_edges(src, dst, nt, nf, ns, g, k, n_body):
    s_r = src.reshape(ns, n_body, k, 1, g)
    d_r = dst.reshape(ns, n_body, k, 1, g)
    per_f = [jnp.concatenate([s_r + f * nt, d_r], axis=3) for f in range(nf)]
    return jnp.stack(per_f).reshape(nf, ns, n_body, 2 * k, g)


def kernel(X, edge_index, train_sample, top_k_indices, degrees, page_ranks, params):
    p = params
    N, D = X.shape
    K = top_k_indices.shape[1]
    nl = jnp.concatenate([train_sample[:, 0], train_sample[:, 1]])
    Nl = nl.shape[0]
    B = 128 if Nl % 128 == 0 else Nl
    BM = 1000 if N % 1000 == 0 else N
    NT = ((N + 127) // 128) * 128

    K_all, V_all = pl.pallas_call(
        _kv_kernel,
        grid=(N // BM,),
        in_specs=[pl.BlockSpec((BM, D), lambda i: (i, 0)),
                  _full_spec((D, D)), _full_spec((1, D)),
                  _full_spec((D, D)), _full_spec((1, D))],
        out_specs=[pl.BlockSpec((BM, D), lambda i: (i, 0)),
                   pl.BlockSpec((BM, D), lambda i: (i, 0))],
        out_shape=[jax.ShapeDtypeStruct((N, D), jnp.float32),
                   jax.ShapeDtypeStruct((N, D), jnp.float32)],
    )(X, p['wk'], _row2(p, 'bk'), p['wv'], _row2(p, 'bv'))

    dp = jnp.stack([degrees[nl], page_ranks[nl]], axis=1)
    fl = top_k_indices[nl].reshape(-1)
    kn_f, vn_f = _sc_gather_pair(K_all, V_all, fl, fl)
    Xs = _sc_gather_one(X, nl)

    w_names = ['deg_w1', 'deg_b1', 'deg_w2', 'deg_b2',
               'pr_w1', 'pr_b1', 'pr_w2', 'pr_b2',
               'pe_w1', 'pe_b1', 'pe_w2', 'pe_b2',
               'wq', 'bq', 'wk', 'bk', 'wv', 'bv', 'wo', 'bo',
               'ln1_g', 'ln1_b', 'ff_w1', 'ff_b1', 'ff_w2', 'ff_b2',
               'ln2_g', 'ln2_b']
    w_vals = [_row2(p, n) for n in w_names]
    cls = pl.pallas_call(
        _encode_kernel,
        grid=(Nl // B,),
        in_specs=[pl.BlockSpec((B, D), lambda i: (i, 0)),
                  pl.BlockSpec((B, 2), lambda i: (i, 0)),
                  pl.BlockSpec((B * K, D), lambda i: (i, 0)),
                  pl.BlockSpec((B * K, D), lambda i: (i, 0))]
                 + [_full_spec(w.shape) for w in w_vals],
        out_specs=pl.BlockSpec((B, D), lambda i: (i, 0)),
        out_shape=jax.ShapeDtypeStruct((Nl, D), jnp.float32),
    )(Xs, dp, kn_f, vn_f, *w_vals)

    Xn = X.at[nl].set(cls)

    src, dst = edge_index[0], edge_index[1]
    E = src.shape[0]
    H1 = p['gcn1_w'].shape[1]
    H2 = p['gcn2_w'].shape[1]
    NF1 = H1 // _FC
    NF2 = H2 // _FC
    zeros_acc = jnp.zeros((NT, _FC), jnp.float32)
    NC, NS = _sc_info()
    epw, g, kk, n_body = _seg_plan(E, NS)
    er1 = _pack_edges(src, dst, NT, NF1, NS, g, kk, n_body)
    ones_gfc = jnp.ones((g, _FC), jnp.float32) + kn_f[0, 0].astype(jnp.float32) * 0.0
    cnt = _sc_count(er1, ones_gfc, zeros_acc,
                    nt=NT, fc=_FC, g=g, k=kk, n_body=n_body)
    deg = cnt[:N, 0] + 1.0
    dinv = jax.lax.rsqrt(deg)
    di2 = dinv[:, None]

    z1 = pl.pallas_call(
        _z1_kernel,
        grid=(N // BM, NF1),
        in_specs=[pl.BlockSpec((BM, D), lambda i, f: (i, 0)),
                  pl.BlockSpec((D, _FC), lambda i, f: (0, f)),
                  pl.BlockSpec((BM, 1), lambda i, f: (i, 0))],
        out_specs=pl.BlockSpec((1, BM, _FC), lambda i, f: (f, i, 0)),
        out_shape=jax.ShapeDtypeStruct((NF1, NT, _FC), jnp.float32),
    )(Xn, p['gcn1_w'], di2)

    p1 = _sc_segsum(z1.reshape(NF1 * NT, _FC), er1, zeros_acc,
                    nt=NT, nf=NF1, fc=_FC, g=g, k=kk, n_body=n_body)

    z2 = pl.pallas_call(
        functools.partial(_mid_kernel, nf=NF1),
        grid=(N // BM, NF2),
        in_specs=[pl.BlockSpec((NF1, BM, _FC), lambda i, f: (0, i, 0)),
                  pl.BlockSpec((NF1, BM, _FC), lambda i, f: (0, i, 0)),
                  pl.BlockSpec((BM, 1), lambda i, f: (i, 0)),
                  pl.BlockSpec((H1, _FC), lambda i, f: (0, f))],
        out_specs=pl.BlockSpec((1, BM, _FC), lambda i, f: (f, i, 0)),
        out_shape=jax.ShapeDtypeStruct((NF2, NT, _FC), jnp.float32),
    )(p1, z1, di2, p['gcn2_w'])

    er2 = er1 if NF2 == NF1 else _pack_edges(src, dst, NT, NF2, NS, g, kk, n_body)
    p2 = _sc_segsum(z2.reshape(NF2 * NT, _FC), er2, zeros_acc,
                    nt=NT, nf=NF2, fc=_FC, g=g, k=kk, n_body=n_body)

    H3 = p['tf1_w'].shape[1]
    OUT = p['tf2_w'].shape[1]
    tf, tg = pl.pallas_call(
        functools.partial(_towers_kernel, nf=NF2),
        grid=(N // BM,),
        in_specs=[pl.BlockSpec((NF2, BM, _FC), lambda i: (0, i, 0)),
                  pl.BlockSpec((NF2, BM, _FC), lambda i: (0, i, 0)),
                  pl.BlockSpec((BM, 1), lambda i: (i, 0)),
                  _full_spec((H2, H3)), _full_spec((1, H3)),
                  _full_spec((H3, OUT)), _full_spec((1, OUT)),
                  _full_spec((H2, H3)), _full_spec((1, H3)),
                  _full_spec((H3, OUT)), _full_spec((1, OUT))],
        out_specs=[pl.BlockSpec((BM, OUT), lambda i: (i, 0)),
                   pl.BlockSpec((BM, OUT), lambda i: (i, 0))],
        out_shape=[jax.ShapeDtypeStruct((N, OUT), jnp.float32),
                   jax.ShapeDtypeStruct((N, OUT), jnp.float32)],
    )(p2, z2, di2,
      p['tf1_w'], _row2(p, 'tf1_b'), p['tf2_w'], _row2(p, 'tf2_b'),
      p['tg1_w'], _row2(p, 'tg1_b'), p['tg2_w'], _row2(p, 'tg2_b'))

    ia = jnp.asarray(train_sample[:, 0])
    ib = jnp.asarray(train_sample[:, 1])
    tfa, tgb = _sc_gather_pair(tf, tg, ia, ib)
    T = ia.shape[0]
    BT = 1024 if T % 1024 == 0 else T
    pred = pl.pallas_call(
        _pred_kernel,
        grid=(T // BT,),
        in_specs=[pl.BlockSpec((BT, OUT), lambda i: (i, 0)),
                  pl.BlockSpec((BT, OUT), lambda i: (i, 0))],
        out_specs=pl.BlockSpec((BT, 1), lambda i: (i, 0)),
        out_shape=jax.ShapeDtypeStruct((T, 1), jnp.float32),
    )(tfa, tgb)
    return pred


# speedup vs baseline: 1.4402x; 1.0768x over previous
import functools

import jax
import jax.numpy as jnp
import numpy as np
from jax import lax
from jax.experimental import pallas as pl
from jax.experimental.pallas import tpu as pltpu
from jax.experimental.pallas import tpu_sc as plsc

_NH = 8
_FC = 128


def _full_spec(shape):
    nd = len(shape)
    return pl.BlockSpec(shape, lambda *a, _nd=nd: (0,) * _nd)


def _ln(x, g, b):
    m = jnp.mean(x, axis=-1, keepdims=True)
    v = jnp.mean((x - m) ** 2, axis=-1, keepdims=True)
    return (x - m) * jax.lax.rsqrt(v + 1e-5) * g + b


def _dot(a, b):
    return jnp.dot(a, b, preferred_element_type=jnp.float32)


def _dotb(a, b):
    return jnp.dot(a.astype(jnp.bfloat16), b.astype(jnp.bfloat16),
                   preferred_element_type=jnp.float32)


def _encode_kernel(xs_ref, dp_ref, kn_ref, vn_ref,
                   wd1_ref, bd1_ref, wd2_ref, bd2_ref,
                   wp1_ref, bp1_ref, wp2_ref, bp2_ref,
                   we1_ref, be1_ref, we2_ref, be2_ref,
                   wq_ref, bq_ref, wk_ref, bk_ref, wv_ref, bv_ref,
                   wo_ref, bo_ref, ln1g_ref, ln1b_ref,
                   fw1_ref, fb1_ref, fw2_ref, fb2_ref,
                   ln2g_ref, ln2b_ref, o_ref):
    xs = xs_ref[...]
    B, D = xs.shape
    K = kn_ref.shape[0] // B
    dh = D // _NH

    deg = dp_ref[:, 0:1]
    prk = dp_ref[:, 1:2]
    de = _dotb(jax.nn.relu(deg * wd1_ref[...] + bd1_ref[...]), wd2_ref[...]) + bd2_ref[...]
    pe = _dotb(jax.nn.relu(prk * wp1_ref[...] + bp1_ref[...]), wp2_ref[...]) + bp2_ref[...]

    we1 = we1_ref[...]
    h = (_dotb(xs, we1[0:D]) + _dotb(de, we1[D:2 * D])
         + _dotb(pe, we1[2 * D:3 * D]) + be1_ref[...])
    xe = _dotb(jax.nn.relu(h), we2_ref[...]) + be2_ref[...]

    q0 = _dot(xe, wq_ref[...]) + bq_ref[...]
    k0 = _dot(xe, wk_ref[...]) + bk_ref[...]
    v0 = _dot(xe, wv_ref[...]) + bv_ref[...]
    kn = kn_ref[...].astype(jnp.float32).reshape(B, K, D)
    vn = vn_ref[...].astype(jnp.float32).reshape(B, K, D)

    hm = (jax.lax.broadcasted_iota(jnp.int32, (D, _NH), 0) // dh
          == jax.lax.broadcasted_iota(jnp.int32, (D, _NH), 1)).astype(jnp.float32)

    scale = 1.0 / np.sqrt(dh)
    prod = kn * q0[:, None, :]
    sn = _dot(prod.reshape(B * K, D), hm).reshape(B, K, _NH) * scale
    s0 = _dot(q0 * k0, hm) * scale
    m = jnp.maximum(jnp.max(sn, axis=1), s0)
    en = jnp.exp(sn - m[:, None, :])
    e0 = jnp.exp(s0 - m)
    den = e0 + jnp.sum(en, axis=1)
    en_x = _dot(en.reshape(B * K, _NH), hm.T).reshape(B, K, D)
    num = jnp.sum(en_x * vn, axis=1) + _dot(e0, hm.T) * v0
    ctx = num / _dot(den, hm.T)
    att = _dot(ctx, wo_ref[...]) + bo_ref[...]

    x1 = _ln(xe + att, ln1g_ref[...], ln1b_ref[...])
    ff = _dotb(jax.nn.relu(_dotb(x1, fw1_ref[...]) + fb1_ref[...]), fw2_ref[...]) + fb2_ref[...]
    o_ref[...] = _ln(x1 + ff, ln2g_ref[...], ln2b_ref[...])


def _kv_kernel(x_ref, wk_ref, bk_ref, wv_ref, bv_ref, ko_ref, vo_ref):
    x = x_ref[...]
    ko_ref[...] = _dot(x, wk_ref[...]) + bk_ref[...]
    vo_ref[...] = _dot(x, wv_ref[...]) + bv_ref[...]


def _z1_kernel(x_ref, w_ref, di_ref, o_ref):
    o_ref[0] = _dotb(x_ref[...], w_ref[...]) * di_ref[...]


def _mid_kernel(p_ref, z_ref, di_ref, w_ref, o_ref, *, nf):
    di = di_ref[...]
    cols = [p_ref[f] + z_ref[f] for f in range(nf)]
    h = jnp.concatenate(cols, axis=-1) * di
    h = jnp.where(h > 0, h, jnp.exp(jnp.minimum(h, 0.0)) - 1.0)
    o_ref[0] = _dotb(h, w_ref[...]) * di


def _towers_kernel(p_ref, z_ref, di_ref,
                   f1w_ref, f1b_ref, f2w_ref, f2b_ref,
                   g1w_ref, g1b_ref, g2w_ref, g2b_ref,
                   tf_ref, tg_ref, *, nf):
    di = di_ref[...]
    cols = [p_ref[f] + z_ref[f] for f in range(nf)]
    emb = jnp.concatenate(cols, axis=-1) * di

    def lrelu(x):
        return jnp.where(x > 0, x, 0.01 * x)

    a = lrelu(_dotb(emb, f1w_ref[...]) + f1b_ref[...])
    tf_ref[...] = lrelu(_dotb(a, f2w_ref[...]) + f2b_ref[...])
    b = lrelu(_dotb(emb, g1w_ref[...]) + g1b_ref[...])
    tg_ref[...] = lrelu(_dotb(b, g2w_ref[...]) + g2b_ref[...])


def _pred_kernel(a_ref, b_ref, o_ref):
    o_ref[...] = jnp.sum(a_ref[...] * b_ref[...], axis=1, keepdims=True)


def _sc_info():
    info = plsc.get_sparse_core_info()
    return info.num_cores, info.num_subcores


def _seg_plan(E, ns):
    epw = E // ns
    for g, k in ((40, 5), (48, 5), (64, 5), (80, 4), (80, 3), (80, 2),
                 (64, 4), (40, 4), (80, 1), (40, 2), (16, 5), (8, 1)):
        if k * g <= 350 and epw % (g * k) == 0:
            return epw, g, k, epw // (g * k)
    return epw, 0, 0, 0


def _sc_segsum(zflat, er, zeros, *, nt, nf, fc, g, k, n_body):
    NC, NS = _sc_info()
    rps = nt // NS
    mesh = plsc.VectorSubcoreMesh(core_axis_name="c", subcore_axis_name="s")

    @functools.partial(
        pl.kernel, mesh=mesh,
        out_type=jax.ShapeDtypeStruct((nf, nt, fc), jnp.float32),
        scratch_types=[
            pltpu.VMEM((2 * k, g), jnp.int32),
            pltpu.VMEM((k, g, fc), jnp.float32),
            pltpu.VMEM_SHARED((nt, fc), jnp.float32),
            pltpu.SemaphoreType.DMA,
            pltpu.SemaphoreType.DMA,
        ],
    )
    def seg(z_hbm, er_hbm, zero_hbm, out_hbm, ev, rows, acc, sem_g, sem_s):
        cid = lax.axis_index("c")
        sid = lax.axis_index("s")
        for f in range(nf):
            @pl.when((f % NC) == cid)
            def _():
                pltpu.sync_copy(zero_hbm.at[pl.ds(sid * rps, rps)],
                                acc.at[pl.ds(sid * rps, rps)])
                plsc.subcore_barrier()

                def body(i, carry):
                    pltpu.sync_copy(er_hbm.at[f, sid, i], ev)
                    hg = []
                    for s in range(k):
                        hg.append(pltpu.async_copy(
                            z_hbm.at[ev.at[2 * s]], rows.at[s], sem_g))
                    hs = []
                    for s in range(k):
                        hg[s].wait()
                        hs.append(pltpu.async_copy(
                            rows.at[s], acc.at[ev.at[2 * s + 1]], sem_s,
                            add=True))
                    for h in hs:
                        h.wait()
                    return carry

                lax.fori_loop(0, n_body, body, 0)
                plsc.subcore_barrier()
                pltpu.sync_copy(acc.at[pl.ds(sid * rps, rps)],
                                out_hbm.at[f, pl.ds(sid * rps, rps)])

    return seg(zflat, er, zeros)


def _sc_count(er, ones_gfc, zeros, *, nt, fc, g, k, n_body):
    NC, NS = _sc_info()
    rps = nt // NS
    mesh = plsc.VectorSubcoreMesh(core_axis_name="c", subcore_axis_name="s")

    @functools.partial(
        pl.kernel, mesh=mesh,
        out_type=jax.ShapeDtypeStruct((nt, fc), jnp.float32),
        scratch_types=[
            pltpu.VMEM((2 * k, g), jnp.int32),
            pltpu.VMEM((g, fc), jnp.float32),
            pltpu.VMEM_SHARED((nt, fc), jnp.float32),
            pltpu.SemaphoreType.DMA,
        ],
    )
    def ck(er_hbm, ones_hbm, zero_hbm, out_hbm, ev, onev, acc, sem_s):
        cid = lax.axis_index("c")
        sid = lax.axis_index("s")

        @pl.when(cid == 0)
        def _():
            pltpu.sync_copy(ones_hbm, onev)
            pltpu.sync_copy(zero_hbm.at[pl.ds(sid * rps, rps)],
                            acc.at[pl.ds(sid * rps, rps)])
            plsc.subcore_barrier()

            def body(i, carry):
                pltpu.sync_copy(er_hbm.at[0, sid, i], ev)
                hs = []
                for s in range(k):
                    hs.append(pltpu.async_copy(
                        onev, acc.at[ev.at[2 * s + 1]], sem_s, add=True))
                for h in hs:
                    h.wait()
                return carry

            lax.fori_loop(0, n_body, body, 0)
            plsc.subcore_barrier()
            pltpu.sync_copy(acc.at[pl.ds(sid * rps, rps)],
                            out_hbm.at[pl.ds(sid * rps, rps)])

    return ck(er, ones_gfc, zeros)


def _gather_plan(B, nw):
    bpw = B // nw
    for g in range(128, 0, -8):
        if bpw % g == 0:
            return bpw, g, bpw // g
    return bpw, 0, 0


def _sc_gather_pair(tab1, tab2, idx1, idx2):
    NC, NS = _sc_info()
    NW = NC * NS
    B = idx1.shape[0]
    D1, D2 = tab1.shape[1], tab2.shape[1]
    bpw, g, n_iter = _gather_plan(B, NW)
    dt1, dt2 = tab1.dtype, tab2.dtype
    mesh = plsc.VectorSubcoreMesh(core_axis_name="c", subcore_axis_name="s")

    @functools.partial(
        pl.kernel, mesh=mesh,
        out_type=[jax.ShapeDtypeStruct((B, D1), dt1),
                  jax.ShapeDtypeStruct((B, D2), dt2)],
        scratch_types=[
            pltpu.VMEM((g,), jnp.int32),
            pltpu.VMEM((g,), jnp.int32),
            pltpu.VMEM((g, D1), dt1),
            pltpu.VMEM((g, D2), dt2),
            pltpu.SemaphoreType.DMA,
            pltpu.SemaphoreType.DMA,
        ],
    )
    def gk(t1, t2, i1_hbm, i2_hbm, o1, o2, iv1, iv2, r1, r2, sem1, sem2):
        wid = lax.axis_index("s") * NC + lax.axis_index("c")

        def body(it, carry):
            base = wid * bpw + it * g
            pltpu.sync_copy(i1_hbm.at[pl.ds(base, g)], iv1)
            pltpu.sync_copy(i2_hbm.at[pl.ds(base, g)], iv2)
            h1 = pltpu.async_copy(t1.at[iv1], r1, sem1)
            h2 = pltpu.async_copy(t2.at[iv2], r2, sem2)
            h1.wait()
            pltpu.sync_copy(r1, o1.at[pl.ds(base, g)])
            h2.wait()
            pltpu.sync_copy(r2, o2.at[pl.ds(base, g)])
            return carry

        lax.fori_loop(0, n_iter, body, 0)

    return gk(tab1, tab2, idx1, idx2)


def _sc_gather_one(tab, idx):
    NC, NS = _sc_info()
    NW = NC * NS
    B = idx.shape[0]
    Dt = tab.shape[1]
    bpw, g, n_iter = _gather_plan(B, NW)
    mesh = plsc.VectorSubcoreMesh(core_axis_name="c", subcore_axis_name="s")

    @functools.partial(
        pl.kernel, mesh=mesh,
        out_type=jax.ShapeDtypeStruct((B, Dt), jnp.float32),
        scratch_types=[
            pltpu.VMEM((g,), jnp.int32),
            pltpu.VMEM((g, Dt), jnp.float32),
            pltpu.SemaphoreType.DMA,
        ],
    )
    def gk(t, i_hbm, o, iv, r, sem):
        wid = lax.axis_index("s") * NC + lax.axis_index("c")

        def body(it, carry):
            base = wid * bpw + it * g
            pltpu.sync_copy(i_hbm.at[pl.ds(base, g)], iv)
            pltpu.async_copy(t.at[iv], r, sem).wait()
            pltpu.sync_copy(r, o.at[pl.ds(base, g)])
            return carry

        lax.fori_loop(0, n_iter, body, 0)

    return gk(tab, idx)


def _row2(p, name):
    w = p[name]
    return w.reshape(1, -1) if w.ndim == 1 else w


def _pack_edges(src, dst, nt, nf, ns, g, k, n_body):
    s_r = src.reshape(ns, n_body, k, 1, g)
    d_r = dst.reshape(ns, n_body, k, 1, g)
    per_f = [jnp.concatenate([s_r + f * nt, d_r], axis=3) for f in range(nf)]
    return jnp.stack(per_f).reshape(nf, ns, n_body, 2 * k, g)


def kernel(X, edge_index, train_sample, top_k_indices, degrees, page_ranks, params):
    p = params
    N, D = X.shape
    K = top_k_indices.shape[1]
    nl = jnp.concatenate([train_sample[:, 0], train_sample[:, 1]])
    Nl = nl.shape[0]
    B = 128 if Nl % 128 == 0 else Nl
    BM = 1000 if N % 1000 == 0 else N
    NT = ((N + 127) // 128) * 128

    K_all, V_all = pl.pallas_call(
        _kv_kernel,
        grid=(N // BM,),
        in_specs=[pl.BlockSpec((BM, D), lambda i: (i, 0)),
                  _full_spec((D, D)), _full_spec((1, D)),
                  _full_spec((D, D)), _full_spec((1, D))],
        out_specs=[pl.BlockSpec((BM, D), lambda i: (i, 0)),
                   pl.BlockSpec((BM, D), lambda i: (i, 0))],
        out_shape=[jax.ShapeDtypeStruct((N, D), jnp.float32),
                   jax.ShapeDtypeStruct((N, D), jnp.float32)],
    )(X, p['wk'], _row2(p, 'bk'), p['wv'], _row2(p, 'bv'))

    dp = jnp.stack([degrees[nl], page_ranks[nl]], axis=1)
    fl = top_k_indices[nl].reshape(-1)
    kn_f, vn_f = _sc_gather_pair(K_all, V_all, fl, fl)
    Xs = _sc_gather_one(X, nl)

    w_names = ['deg_w1', 'deg_b1', 'deg_w2', 'deg_b2',
               'pr_w1', 'pr_b1', 'pr_w2', 'pr_b2',
               'pe_w1', 'pe_b1', 'pe_w2', 'pe_b2',
               'wq', 'bq', 'wk', 'bk', 'wv', 'bv', 'wo', 'bo',
               'ln1_g', 'ln1_b', 'ff_w1', 'ff_b1', 'ff_w2', 'ff_b2',
               'ln2_g', 'ln2_b']
    w_vals = [_row2(p, n) for n in w_names]
    cls = pl.pallas_call(
        _encode_kernel,
        grid=(Nl // B,),
        in_specs=[pl.BlockSpec((B, D), lambda i: (i, 0)),
                  pl.BlockSpec((B, 2), lambda i: (i, 0)),
                  pl.BlockSpec((B * K, D), lambda i: (i, 0)),
                  pl.BlockSpec((B * K, D), lambda i: (i, 0))]
                 + [_full_spec(w.shape) for w in w_vals],
        out_specs=pl.BlockSpec((B, D), lambda i: (i, 0)),
        out_shape=jax.ShapeDtypeStruct((Nl, D), jnp.float32),
    )(Xs, dp, kn_f, vn_f, *w_vals)

    Xn = X.at[nl].set(cls)

    src, dst = edge_index[0], edge_index[1]
    E = src.shape[0]
    H1 = p['gcn1_w'].shape[1]
    H2 = p['gcn2_w'].shape[1]
    NF1 = H1 // _FC
    NF2 = H2 // _FC
    zeros_acc = jnp.zeros((NT, _FC), jnp.float32)
    NC, NS = _sc_info()
    epw, g, kk, n_body = _seg_plan(E, NS)
    er1 = _pack_edges(src, dst, NT, NF1, NS, g, kk, n_body)
    ones_gfc = jnp.ones((g, _FC), jnp.float32) + kn_f[0, 0].astype(jnp.float32) * 0.0
    cnt = _sc_count(er1, ones_gfc, zeros_acc,
                    nt=NT, fc=_FC, g=g, k=kk, n_body=n_body)
    deg = cnt[:N, 0] + 1.0
    dinv = jax.lax.rsqrt(deg)
    di2 = dinv[:, None]

    z1 = pl.pallas_call(
        _z1_kernel,
        grid=(N // BM, NF1),
        in_specs=[pl.BlockSpec((BM, D), lambda i, f: (i, 0)),
                  pl.BlockSpec((D, _FC), lambda i, f: (0, f)),
                  pl.BlockSpec((BM, 1), lambda i, f: (i, 0))],
        out_specs=pl.BlockSpec((1, BM, _FC), lambda i, f: (f, i, 0)),
        out_shape=jax.ShapeDtypeStruct((NF1, NT, _FC), jnp.float32),
    )(Xn, p['gcn1_w'], di2)

    p1 = _sc_segsum(z1.reshape(NF1 * NT, _FC), er1, zeros_acc,
                    nt=NT, nf=NF1, fc=_FC, g=g, k=kk, n_body=n_body)

    z2 = pl.pallas_call(
        functools.partial(_mid_kernel, nf=NF1),
        grid=(N // BM, NF2),
        in_specs=[pl.BlockSpec((NF1, BM, _FC), lambda i, f: (0, i, 0)),
                  pl.BlockSpec((NF1, BM, _FC), lambda i, f: (0, i, 0)),
                  pl.BlockSpec((BM, 1), lambda i, f: (i, 0)),
                  pl.BlockSpec((H1, _FC), lambda i, f: (0, f))],
        out_specs=pl.BlockSpec((1, BM, _FC), lambda i, f: (f, i, 0)),
        out_shape=jax.ShapeDtypeStruct((NF2, NT, _FC), jnp.float32),
    )(p1, z1, di2, p['gcn2_w'])

    er2 = er1 if NF2 == NF1 else _pack_edges(src, dst, NT, NF2, NS, g, kk, n_body)
    p2 = _sc_segsum(z2.reshape(NF2 * NT, _FC), er2, zeros_acc,
                    nt=NT, nf=NF2, fc=_FC, g=g, k=kk, n_body=n_body)

    H3 = p['tf1_w'].shape[1]
    OUT = p['tf2_w'].shape[1]
    tf, tg = pl.pallas_call(
        functools.partial(_towers_kernel, nf=NF2),
        grid=(N // BM,),
        in_specs=[pl.BlockSpec((NF2, BM, _FC), lambda i: (0, i, 0)),
                  pl.BlockSpec((NF2, BM, _FC), lambda i: (0, i, 0)),
                  pl.BlockSpec((BM, 1), lambda i: (i, 0)),
                  _full_spec((H2, H3)), _full_spec((1, H3)),
                  _full_spec((H3, OUT)), _full_spec((1, OUT)),
                  _full_spec((H2, H3)), _full_spec((1, H3)),
                  _full_spec((H3, OUT)), _full_spec((1, OUT))],
        out_specs=[pl.BlockSpec((BM, OUT), lambda i: (i, 0)),
                   pl.BlockSpec((BM, OUT), lambda i: (i, 0))],
        out_shape=[jax.ShapeDtypeStruct((N, OUT), jnp.float32),
                   jax.ShapeDtypeStruct((N, OUT), jnp.float32)],
    )(p2, z2, di2,
      p['tf1_w'], _row2(p, 'tf1_b'), p['tf2_w'], _row2(p, 'tf2_b'),
      p['tg1_w'], _row2(p, 'tg1_b'), p['tg2_w'], _row2(p, 'tg2_b'))

    ia = jnp.asarray(train_sample[:, 0])
    ib = jnp.asarray(train_sample[:, 1])
    tfa, tgb = _sc_gather_pair(tf, tg, ia, ib)
    T = ia.shape[0]
    BT = 1024 if T % 1024 == 0 else T
    pred = pl.pallas_call(
        _pred_kernel,
        grid=(T // BT,),
        in_specs=[pl.BlockSpec((BT, OUT), lambda i: (i, 0)),
                  pl.BlockSpec((BT, OUT), lambda i: (i, 0))],
        out_specs=pl.BlockSpec((BT, 1), lambda i: (i, 0)),
        out_shape=jax.ShapeDtypeStruct((T, 1), jnp.float32),
    )(tfa, tgb)
    return pred
